# Initial kernel scaffold; baseline (speedup 1.0000x reference)
import jax
import jax.numpy as jnp
from jax.experimental import pallas as pl


def kernel(x_seq, edge_index, W_i, W_f, W_c, W_o, Theta_i, Theta_f, Theta_c, Theta_o, bc_i, bc_f, bc_c, bc_o, w_c_i, w_c_f, w_c_o, b_i, b_f, b_c, b_o, W_lin, b_lin):
    raise NotImplementedError("write your pallas kernel here")


# speedup vs baseline: 3.8320x; 3.8320x over previous
import functools

import jax
import jax.numpy as jnp
from jax.experimental import pallas as pl
from jax.experimental.pallas import tpu as pltpu

_T = 8
_F = 3
_HD = 64


def _gclstm_kernel(x_ref, w_all_ref, th_all_ref, par_ref, wl_ref, out_ref,
                   acc_ref, *, n_valid, n_blocks, blk):
    i = pl.program_id(0)

    @pl.when(i == 0)
    def _():
        acc_ref[...] = jnp.zeros_like(acc_ref)

    w_all = w_all_ref[...]
    th_all = th_all_ref[...]
    par = par_ref[...]
    bi = par[:, 0:1]
    bf = par[:, 1:2]
    bc = par[:, 2:3]
    bo = par[:, 3:4]
    wci = par[:, 4:5]
    wcf = par[:, 5:6]
    wco = par[:, 6:7]

    h = jnp.zeros((_HD, blk), jnp.float32)
    c = jnp.zeros((_HD, blk), jnp.float32)
    for t in range(_T):
        x_t = x_ref[_F * t:_F * t + _F, :]
        mm = jnp.dot(w_all, x_t, preferred_element_type=jnp.float32)
        if t > 0:
            mm = mm + jnp.dot(th_all, h, preferred_element_type=jnp.float32)
        gi = jax.nn.sigmoid(mm[0:_HD] + bi + wci * c)
        gt = jnp.tanh(mm[2 * _HD:3 * _HD] + bc)
        if t > 0:
            gf = jax.nn.sigmoid(mm[_HD:2 * _HD] + bf + wcf * c)
            c = gf * c + gi * gt
        else:
            c = gi * gt
        go = jax.nn.sigmoid(mm[3 * _HD:4 * _HD] + bo + wco * c)
        h = go * jnp.tanh(c)

    lane = jax.lax.broadcasted_iota(jnp.int32, (_HD, blk), 1)
    mask = (i * blk + lane) < n_valid
    acc_ref[...] += jnp.where(mask, h, 0.0)

    @pl.when(i == n_blocks - 1)
    def _():
        g_row = jnp.sum(acc_ref[...], axis=1).reshape(1, _HD) * (1.0 / n_valid)
        wl = wl_ref[...]
        logits = jnp.dot(g_row, wl[:, 0:2], preferred_element_type=jnp.float32)
        out_ref[...] = logits


def kernel(x_seq, edge_index, W_i, W_f, W_c, W_o, Theta_i, Theta_f, Theta_c,
           Theta_o, bc_i, bc_f, bc_c, bc_o, w_c_i, w_c_f, w_c_o, b_i, b_f,
           b_c, b_o, W_lin, b_lin):
    del edge_index
    T, N, F = x_seq.shape
    blk = 2048
    n_blocks = pl.cdiv(N, blk)

    x2 = x_seq.transpose(0, 2, 1).reshape(T * F, N)

    w_all = jnp.concatenate([W_i, W_f, W_c, W_o], axis=1).T
    th_all = jnp.concatenate([Theta_i, Theta_f, Theta_c, Theta_o], axis=1).T
    par = jnp.stack([bc_i + b_i[0], bc_f + b_f[0], bc_c + b_c[0],
                     bc_o + b_o[0], w_c_i[0], w_c_f[0], w_c_o[0],
                     jnp.zeros((_HD,), jnp.float32)], axis=1)
    wl = jnp.concatenate([W_lin, jnp.zeros((_HD, 1), jnp.float32)], axis=1)

    out = pl.pallas_call(
        functools.partial(_gclstm_kernel, n_valid=N, n_blocks=n_blocks,
                          blk=blk),
        grid=(n_blocks,),
        in_specs=[
            pl.BlockSpec((T * F, blk), lambda i: (0, i)),
            pl.BlockSpec((4 * _HD, F), lambda i: (0, 0)),
            pl.BlockSpec((4 * _HD, _HD), lambda i: (0, 0)),
            pl.BlockSpec((_HD, 8), lambda i: (0, 0)),
            pl.BlockSpec((_HD, F), lambda i: (0, 0)),
        ],
        out_specs=pl.BlockSpec((1, 2), lambda i: (0, 0)),
        out_shape=jax.ShapeDtypeStruct((1, 2), jnp.float32),
        scratch_shapes=[pltpu.VMEM((_HD, blk), jnp.float32)],
        compiler_params=pltpu.CompilerParams(
            dimension_semantics=("arbitrary",)),
    )(x2, w_all, th_all, par, wl)
    return out + b_lin
